# baseline (speedup 1.0000x reference)
import functools

import jax
import jax.numpy as jnp
from jax import lax
from jax.experimental import pallas as pl
from jax.experimental.pallas import tpu as pltpu
from jax.experimental.pallas import tpu_sc as plsc

D = 128
N_NODES = 100000
N0 = 10000
N1 = 2000
N2 = 1024

_INFO = plsc.get_sparse_core_info()
NC = _INFO.num_cores
NS = _INFO.num_subcores
NW = NC * NS

_mesh = plsc.VectorSubcoreMesh(core_axis_name="c", subcore_axis_name="s")

FAST_CORE = 1
S_FAST = 448
S_SLOW = 176
S_EXTRA = 16
S_SLOW_BASE = 16 * S_FAST
assert 16 * (S_FAST + S_SLOW) + S_EXTRA == N0
D_FAST = 96
D_SLOW = 24
D_EXTRA = 80
D_SLOW_BASE = 16 * D_FAST
assert 16 * (D_FAST + D_SLOW) + D_EXTRA == N1
SBUF = S_FAST + S_EXTRA
DBUF = D_FAST + D_EXTRA


def _chunks(total, size):
    out, off = [], 0
    while off < total:
        c = min(size, total - off)
        out.append((off, c))
        off += c
    return out


@functools.partial(
    pl.kernel,
    out_type=[
        jax.ShapeDtypeStruct((N0, D), jnp.float32),
        jax.ShapeDtypeStruct((N1, D), jnp.float32),
    ],
    mesh=_mesh,
    scratch_types=[
        pltpu.VMEM((SBUF,), jnp.int32),
        pltpu.VMEM((DBUF,), jnp.int32),
        pltpu.VMEM((SBUF,), jnp.int32),
        pltpu.VMEM((DBUF,), jnp.int32),
        pltpu.VMEM((SBUF, D), jnp.float32),
        pltpu.VMEM((DBUF, D), jnp.float32),
        pltpu.VMEM_SHARED((N0,), jnp.int32),
        pltpu.SemaphoreType.DMA,
        pltpu.SemaphoreType.DMA,
        pltpu.SemaphoreType.DMA,
        pltpu.SemaphoreType.DMA,
    ],
)
def _gather_l1(raw_hbm, srcn_hbm, s1_hbm, d1_hbm, src1_out, dst1_out,
               s1v, d1v, cs1v, cd1v, rows_v, drows_v, snv_sh,
               isem, csem, rsem, wsem):
    cid = lax.axis_index("c")
    sid = lax.axis_index("s")

    @pl.when(sid == 0)
    def _():
        pltpu.sync_copy(srcn_hbm, snv_sh)

    plsc.subcore_barrier()

    def _ds(off, n):
        if isinstance(off, int):
            return pl.ds(off, n)
        return pl.ds(pl.multiple_of(off, 8), n)

    def _run_spans(s_spans, d_spans, chunk_s, chunk_d):
        ids = []
        for hoff, voff, n in s_spans:
            ids.append(pltpu.async_copy(s1_hbm.at[_ds(hoff, n)],
                                        s1v.at[pl.ds(voff, n)], isem))
        for hoff, voff, n in d_spans:
            ids.append(pltpu.async_copy(d1_hbm.at[_ds(hoff, n)],
                                        d1v.at[pl.ds(voff, n)], isem))
        for dsc in ids:
            dsc.wait()
        cds = []
        for hoff, voff, n in s_spans:
            for coff, cn in _chunks(n, chunk_s):
                cds.append(pltpu.async_copy(
                    snv_sh.at[s1v.at[pl.ds(voff + coff, cn)]],
                    cs1v.at[pl.ds(voff + coff, cn)], csem))
        for hoff, voff, n in d_spans:
            for coff, cn in _chunks(n, chunk_d):
                cds.append(pltpu.async_copy(
                    snv_sh.at[d1v.at[pl.ds(voff + coff, cn)]],
                    cd1v.at[pl.ds(voff + coff, cn)], csem))
        for dsc in cds:
            dsc.wait()
        rds = []
        for hoff, voff, n in s_spans:
            for coff, cn in _chunks(n, chunk_s):
                rds.append(pltpu.async_copy(
                    raw_hbm.at[cs1v.at[pl.ds(voff + coff, cn)]],
                    rows_v.at[pl.ds(voff + coff, cn)], rsem))
        for hoff, voff, n in d_spans:
            for coff, cn in _chunks(n, chunk_d):
                rds.append(pltpu.async_copy(
                    raw_hbm.at[cd1v.at[pl.ds(voff + coff, cn)]],
                    drows_v.at[pl.ds(voff + coff, cn)], rsem))
        for dsc in rds:
            dsc.wait()
        wds = []
        for hoff, voff, n in s_spans:
            wds.append(pltpu.async_copy(
                rows_v.at[pl.ds(voff, n)],
                src1_out.at[pl.ds(hoff, n)], wsem))
        for hoff, voff, n in d_spans:
            wds.append(pltpu.async_copy(
                drows_v.at[pl.ds(voff, n)],
                dst1_out.at[pl.ds(hoff, n)], wsem))
        for dsc in wds:
            dsc.wait()

    s_ex = (S_SLOW_BASE + 16 * S_SLOW, S_FAST, S_EXTRA)
    d_ex = (D_SLOW_BASE + 16 * D_SLOW, D_FAST, D_EXTRA)

    @pl.when((cid == FAST_CORE) & (sid == 0))
    def _():
        _run_spans([(0, 0, S_FAST), s_ex], [(0, 0, D_FAST), d_ex], 112, 96)

    @pl.when((cid == FAST_CORE) & (sid != 0))
    def _():
        _run_spans([(sid * S_FAST, 0, S_FAST)],
                   [(sid * D_FAST, 0, D_FAST)], 112, 96)

    @pl.when(cid != FAST_CORE)
    def _():
        _run_spans([(S_SLOW_BASE + sid * S_SLOW, 0, S_SLOW)],
                   [(D_SLOW_BASE + sid * D_SLOW, 0, D_SLOW)], 88, 24)


L1_KB = 2048
L1_STEPS = 5


def _tc_body(dif1_ref, src_ref, dst_ref, dif2_ref, s2_ref, d2_ref,
             w1a_ref, w1b_ref, w2a_ref, w2b_ref, out_ref, acc_ref):
    k = pl.program_id(0)
    bf16 = jnp.bfloat16

    @pl.when(k == 0)
    def _():
        acc_ref[...] = jnp.zeros_like(acc_ref)

    @pl.when(k < L1_STEPS - 1)
    def _():
        acc_ref[...] += jnp.dot(dif1_ref[...], src_ref[...],
                                preferred_element_type=jnp.float32)

    @pl.when(k == L1_STEPS - 1)
    def _():
        rem = N0 - (L1_STEPS - 1) * L1_KB
        cols = lax.broadcasted_iota(jnp.int32, (N1, L1_KB), 1)
        dif = jnp.where(cols < rem, dif1_ref[...], 0.0)
        srows = lax.broadcasted_iota(jnp.int32, (L1_KB, D), 0)
        src = jnp.where(srows < rem, src_ref[...], 0.0)
        acc = acc_ref[...] + jnp.dot(dif, src,
                                     preferred_element_type=jnp.float32)
        x1 = jnp.maximum(
            jnp.dot(dst_ref[...], w1a_ref[...],
                    preferred_element_type=jnp.float32)
            + jnp.dot(acc, w1b_ref[...],
                      preferred_element_type=jnp.float32),
            0.0)
        x1b = x1.astype(bf16)
        cols_s = lax.broadcasted_iota(jnp.int32, (N1, N1), 1)
        oh_s = (s2_ref[...] == cols_s).astype(bf16)
        src2 = jnp.dot(oh_s, x1b, preferred_element_type=jnp.float32)
        agg2 = jnp.dot(dif2_ref[...].astype(bf16), src2.astype(bf16),
                       preferred_element_type=jnp.float32)
        cols_d = lax.broadcasted_iota(jnp.int32, (N2, N1), 1)
        oh_d = (d2_ref[...] == cols_d).astype(bf16)
        dst2 = jnp.dot(oh_d, x1b, preferred_element_type=jnp.float32)
        out_ref[...] = (
            jnp.dot(dst2.astype(bf16), w2a_ref[...].astype(bf16),
                    preferred_element_type=jnp.float32)
            + jnp.dot(agg2.astype(bf16), w2b_ref[...].astype(bf16),
                      preferred_element_type=jnp.float32))


def _tc_fused(dif1, src1, dst1, dif2, s2, d2, w1a, w1b, w2a, w2b):
    return pl.pallas_call(
        _tc_body,
        grid=(L1_STEPS,),
        in_specs=[
            pl.BlockSpec((N1, L1_KB), lambda k: (0, k)),
            pl.BlockSpec((L1_KB, D), lambda k: (k, 0)),
            pl.BlockSpec((N1, D), lambda k: (0, 0)),
            pl.BlockSpec((N2, N1), lambda k: (0, 0)),
            pl.BlockSpec((N1, 1), lambda k: (0, 0)),
            pl.BlockSpec((N2, 1), lambda k: (0, 0)),
            pl.BlockSpec((D, D), lambda k: (0, 0)),
            pl.BlockSpec((D, D), lambda k: (0, 0)),
            pl.BlockSpec((D, D), lambda k: (0, 0)),
            pl.BlockSpec((D, D), lambda k: (0, 0)),
        ],
        out_specs=pl.BlockSpec((N2, D), lambda k: (0, 0)),
        out_shape=jax.ShapeDtypeStruct((N2, D), jnp.float32),
        scratch_shapes=[pltpu.VMEM((N1, D), jnp.float32)],
        compiler_params=pltpu.CompilerParams(
            dimension_semantics=("arbitrary",)),
    )(dif1, src1, dst1, dif2, s2, d2, w1a, w1b, w2a, w2b)


def kernel(raw_features, src_nodes, dstsrc2src_l1, dstsrc2dst_l1, dif_mat_l1,
           dstsrc2src_l2, dstsrc2dst_l2, dif_mat_l2, w1, w2):
    i32 = jnp.int32
    src1, dst1 = _gather_l1(raw_features, src_nodes.astype(i32),
                            dstsrc2src_l1.astype(i32),
                            dstsrc2dst_l1.astype(i32))
    s2c = dstsrc2src_l2.astype(i32).reshape(N1, 1)
    d2c = dstsrc2dst_l2.astype(i32).reshape(N2, 1)
    return _tc_fused(dif_mat_l1, src1, dst1, dif_mat_l2, s2c, d2c,
                     w1[:D], w1[D:], w2[:D], w2[D:])


# speedup vs baseline: 1.0447x; 1.0447x over previous
import functools

import jax
import jax.numpy as jnp
from jax import lax
from jax.experimental import pallas as pl
from jax.experimental.pallas import tpu as pltpu
from jax.experimental.pallas import tpu_sc as plsc

D = 128
N_NODES = 100000
N0 = 10000
N1 = 2000
N2 = 1024

_INFO = plsc.get_sparse_core_info()
NC = _INFO.num_cores
NS = _INFO.num_subcores
NW = NC * NS

_mesh = plsc.VectorSubcoreMesh(core_axis_name="c", subcore_axis_name="s")

FAST_CORE = 1
S_FAST = 480
S_SLOW = 144
S_EXTRA = 16
S_SLOW_BASE = 16 * S_FAST
assert 16 * (S_FAST + S_SLOW) + S_EXTRA == N0
D_FAST = 104
D_SLOW = 16
D_EXTRA = 80
D_SLOW_BASE = 16 * D_FAST
assert 16 * (D_FAST + D_SLOW) + D_EXTRA == N1
SBUF = S_FAST + S_EXTRA
DBUF = D_FAST + D_EXTRA


def _chunks(total, size):
    out, off = [], 0
    while off < total:
        c = min(size, total - off)
        out.append((off, c))
        off += c
    return out


@functools.partial(
    pl.kernel,
    out_type=[
        jax.ShapeDtypeStruct((N0, D), jnp.float32),
        jax.ShapeDtypeStruct((N1, D), jnp.float32),
    ],
    mesh=_mesh,
    scratch_types=[
        pltpu.VMEM((SBUF,), jnp.int32),
        pltpu.VMEM((DBUF,), jnp.int32),
        pltpu.VMEM((SBUF,), jnp.int32),
        pltpu.VMEM((DBUF,), jnp.int32),
        pltpu.VMEM((SBUF, D), jnp.float32),
        pltpu.VMEM((DBUF, D), jnp.float32),
        pltpu.VMEM_SHARED((N0,), jnp.int32),
        pltpu.SemaphoreType.DMA,
        pltpu.SemaphoreType.DMA,
        pltpu.SemaphoreType.DMA,
        pltpu.SemaphoreType.DMA,
    ],
)
def _gather_l1(raw_hbm, srcn_hbm, s1_hbm, d1_hbm, src1_out, dst1_out,
               s1v, d1v, cs1v, cd1v, rows_v, drows_v, snv_sh,
               isem, csem, rsem, wsem):
    cid = lax.axis_index("c")
    sid = lax.axis_index("s")

    @pl.when(sid == 0)
    def _():
        pltpu.sync_copy(srcn_hbm, snv_sh)

    plsc.subcore_barrier()

    def _ds(off, n):
        if isinstance(off, int):
            return pl.ds(off, n)
        return pl.ds(pl.multiple_of(off, 8), n)

    def _run_spans(s_spans, d_spans, chunk_s, chunk_d):
        ids = []
        for hoff, voff, n in s_spans:
            ids.append(pltpu.async_copy(s1_hbm.at[_ds(hoff, n)],
                                        s1v.at[pl.ds(voff, n)], isem))
        for hoff, voff, n in d_spans:
            ids.append(pltpu.async_copy(d1_hbm.at[_ds(hoff, n)],
                                        d1v.at[pl.ds(voff, n)], isem))
        for dsc in ids:
            dsc.wait()
        cds = []
        for hoff, voff, n in s_spans:
            for coff, cn in _chunks(n, chunk_s):
                cds.append(pltpu.async_copy(
                    snv_sh.at[s1v.at[pl.ds(voff + coff, cn)]],
                    cs1v.at[pl.ds(voff + coff, cn)], csem))
        for hoff, voff, n in d_spans:
            for coff, cn in _chunks(n, chunk_d):
                cds.append(pltpu.async_copy(
                    snv_sh.at[d1v.at[pl.ds(voff + coff, cn)]],
                    cd1v.at[pl.ds(voff + coff, cn)], csem))
        for dsc in cds:
            dsc.wait()
        rds = []
        for hoff, voff, n in s_spans:
            for coff, cn in _chunks(n, chunk_s):
                rds.append(pltpu.async_copy(
                    raw_hbm.at[cs1v.at[pl.ds(voff + coff, cn)]],
                    rows_v.at[pl.ds(voff + coff, cn)], rsem))
        for hoff, voff, n in d_spans:
            for coff, cn in _chunks(n, chunk_d):
                rds.append(pltpu.async_copy(
                    raw_hbm.at[cd1v.at[pl.ds(voff + coff, cn)]],
                    drows_v.at[pl.ds(voff + coff, cn)], rsem))
        for dsc in rds:
            dsc.wait()
        wds = []
        for hoff, voff, n in s_spans:
            wds.append(pltpu.async_copy(
                rows_v.at[pl.ds(voff, n)],
                src1_out.at[pl.ds(hoff, n)], wsem))
        for hoff, voff, n in d_spans:
            wds.append(pltpu.async_copy(
                drows_v.at[pl.ds(voff, n)],
                dst1_out.at[pl.ds(hoff, n)], wsem))
        for dsc in wds:
            dsc.wait()

    s_ex = (S_SLOW_BASE + 16 * S_SLOW, S_FAST, S_EXTRA)
    d_ex = (D_SLOW_BASE + 16 * D_SLOW, D_FAST, D_EXTRA)

    @pl.when((cid == FAST_CORE) & (sid == 0))
    def _():
        _run_spans([(0, 0, S_FAST), s_ex], [(0, 0, D_FAST), d_ex], 120, 104)

    @pl.when((cid == FAST_CORE) & (sid != 0))
    def _():
        _run_spans([(sid * S_FAST, 0, S_FAST)],
                   [(sid * D_FAST, 0, D_FAST)], 120, 104)

    @pl.when(cid != FAST_CORE)
    def _():
        _run_spans([(S_SLOW_BASE + sid * S_SLOW, 0, S_SLOW)],
                   [(D_SLOW_BASE + sid * D_SLOW, 0, D_SLOW)], 72, 16)


L1_KB = 2048
L1_STEPS = 5


def _tc_body(dif1_ref, src_ref, dst_ref, dif2_ref, s2_ref, d2_ref,
             w1a_ref, w1b_ref, w2a_ref, w2b_ref, out_ref, acc_ref):
    k = pl.program_id(0)
    bf16 = jnp.bfloat16

    @pl.when(k == 0)
    def _():
        acc_ref[...] = jnp.zeros_like(acc_ref)

    @pl.when(k < L1_STEPS - 1)
    def _():
        acc_ref[...] += jnp.dot(dif1_ref[...], src_ref[...],
                                preferred_element_type=jnp.float32)

    @pl.when(k == L1_STEPS - 1)
    def _():
        rem = N0 - (L1_STEPS - 1) * L1_KB
        cols = lax.broadcasted_iota(jnp.int32, (N1, L1_KB), 1)
        dif = jnp.where(cols < rem, dif1_ref[...], 0.0)
        srows = lax.broadcasted_iota(jnp.int32, (L1_KB, D), 0)
        src = jnp.where(srows < rem, src_ref[...], 0.0)
        acc = acc_ref[...] + jnp.dot(dif, src,
                                     preferred_element_type=jnp.float32)
        x1 = jnp.maximum(
            jnp.dot(dst_ref[...], w1a_ref[...],
                    preferred_element_type=jnp.float32)
            + jnp.dot(acc, w1b_ref[...],
                      preferred_element_type=jnp.float32),
            0.0)
        x1b = x1.astype(bf16)
        cols_s = lax.broadcasted_iota(jnp.int32, (N1, N1), 1)
        oh_s = (s2_ref[...] == cols_s).astype(bf16)
        src2 = jnp.dot(oh_s, x1b, preferred_element_type=jnp.float32)
        agg2 = lax.dot_general(
            dif2_ref[...].astype(bf16), src2.astype(bf16),
            (((0,), (0,)), ((), ())),
            preferred_element_type=jnp.float32)
        cols_d = lax.broadcasted_iota(jnp.int32, (N2, N1), 1)
        oh_d = (d2_ref[...] == cols_d).astype(bf16)
        dst2 = jnp.dot(oh_d, x1b, preferred_element_type=jnp.float32)
        out_ref[...] = (
            jnp.dot(dst2.astype(bf16), w2a_ref[...].astype(bf16),
                    preferred_element_type=jnp.float32)
            + jnp.dot(agg2.astype(bf16), w2b_ref[...].astype(bf16),
                      preferred_element_type=jnp.float32))


def _tc_fused(dif1, src1, dst1, dif2, s2, d2, w1a, w1b, w2a, w2b):
    return pl.pallas_call(
        _tc_body,
        grid=(L1_STEPS,),
        in_specs=[
            pl.BlockSpec((N1, L1_KB), lambda k: (0, k)),
            pl.BlockSpec((L1_KB, D), lambda k: (k, 0)),
            pl.BlockSpec((N1, D), lambda k: (0, 0)),
            pl.BlockSpec((N1, N2), lambda k: (0, 0)),
            pl.BlockSpec((N1, 1), lambda k: (0, 0)),
            pl.BlockSpec((N2, 1), lambda k: (0, 0)),
            pl.BlockSpec((D, D), lambda k: (0, 0)),
            pl.BlockSpec((D, D), lambda k: (0, 0)),
            pl.BlockSpec((D, D), lambda k: (0, 0)),
            pl.BlockSpec((D, D), lambda k: (0, 0)),
        ],
        out_specs=pl.BlockSpec((N2, D), lambda k: (0, 0)),
        out_shape=jax.ShapeDtypeStruct((N2, D), jnp.float32),
        scratch_shapes=[pltpu.VMEM((N1, D), jnp.float32)],
        compiler_params=pltpu.CompilerParams(
            dimension_semantics=("arbitrary",)),
    )(dif1, src1, dst1, dif2, s2, d2, w1a, w1b, w2a, w2b)


def kernel(raw_features, src_nodes, dstsrc2src_l1, dstsrc2dst_l1, dif_mat_l1,
           dstsrc2src_l2, dstsrc2dst_l2, dif_mat_l2, w1, w2):
    i32 = jnp.int32
    src1, dst1 = _gather_l1(raw_features, src_nodes.astype(i32),
                            dstsrc2src_l1.astype(i32),
                            dstsrc2dst_l1.astype(i32))
    s2c = dstsrc2src_l2.astype(i32).reshape(N1, 1)
    d2c = dstsrc2dst_l2.astype(i32).reshape(N2, 1)
    return _tc_fused(dif_mat_l1, src1, dst1, dif_mat_l2.T, s2c, d2c,
                     w1[:D], w1[D:], w2[:D], w2[D:])
